# Initial kernel scaffold; baseline (speedup 1.0000x reference)
import jax
import jax.numpy as jnp
from jax.experimental import pallas as pl


def kernel(z, v_in, g_in, b_in, v_out, g_out, b_out, codebooks):
    raise NotImplementedError("write your pallas kernel here")


# speedup vs baseline: 2.1484x; 2.1484x over previous
import jax
import jax.numpy as jnp
from jax.experimental import pallas as pl
from jax.experimental.pallas import tpu as pltpu

B, D, T = 16, 1024, 1024
NCB, K, CD = 8, 1024, 64
TT = 256
TPB = T // TT


def _dot(a, b):
    return jax.lax.dot_general(a, b, (((1,), (0,)), ((), ())),
                               preferred_element_type=jnp.float32)


def _rvq_kernel(z_ref, w_in_ref, b_in_ref, w_out_ref, b_out_ref,
                cbn_ref, cbt_ref, c2_ref,
                codes_ref, latent_ref, loss_ref):
    z0 = z_ref[0]
    res = z0
    kiota = jax.lax.broadcasted_iota(jnp.int32, (K, TT), 0)
    for i in range(NCB):
        z_e = _dot(w_in_ref[i], res) + b_in_ref[i]
        acc = z_e * z_e
        h = CD
        while h > 1:
            h //= 2
            acc = acc[:h] + acc[h:]
        nrm = jnp.sqrt(acc)
        nrm = jnp.maximum(nrm, 1e-12)
        z_en = z_e / nrm
        s = _dot(cbn_ref[i], z_en)
        e = jnp.sum(z_en * z_en, axis=0, keepdims=True)
        dist = (e - 2.0 * s) + c2_ref[i]
        m = jnp.min(dist, axis=0, keepdims=True)
        hit = dist <= m
        idx = jnp.min(jnp.where(hit, kiota, K), axis=0, keepdims=True)
        oh = (kiota == idx).astype(jnp.float32)
        z_q = _dot(cbt_ref[i], oh)
        dq = z_e - z_q
        loss_ref[0, i:i + 1, :] = jnp.sum(dq * dq, axis=0, keepdims=True)
        z_q_out = _dot(w_out_ref[i], z_q) + b_out_ref[i]
        res = res - z_q_out
        codes_ref[0, i:i + 1, :] = idx
    latent_ref[0] = z0 - res


def kernel(z, v_in, g_in, b_in, v_out, g_out, b_out, codebooks):
    f32 = jnp.float32
    w_in_l, w_out_l, cbn_l, c2_l = [], [], [], []
    for i in range(NCB):
        wv = v_in[i]
        wnorm = jnp.sqrt(jnp.sum(wv * wv, axis=1, keepdims=True))
        w_in_l.append(g_in[i][:, None] * wv / wnorm)
        wo = v_out[i]
        wnorm_o = jnp.sqrt(jnp.sum(wo * wo, axis=1, keepdims=True))
        w_out_l.append(g_out[i][:, None] * wo / wnorm_o)
        cb = codebooks[i]
        cn = jnp.sqrt(jnp.sum(cb * cb, axis=1, keepdims=True))
        cbn_i = cb / jnp.maximum(cn, 1e-12)
        cbn_l.append(cbn_i)
        c2_l.append(jnp.sum(cbn_i * cbn_i, axis=1, keepdims=True))
    w_in = jnp.stack(w_in_l)
    w_out = jnp.stack(w_out_l)
    cbn = jnp.stack(cbn_l)
    c2 = jnp.stack(c2_l)
    cbt = jnp.swapaxes(codebooks, 1, 2)
    b_in_r = b_in[:, :, None]
    b_out_r = b_out[:, :, None]

    whole = lambda shape: pl.BlockSpec(shape, lambda j: (0,) * len(shape))
    grid = (B * TPB,)
    codes, latent, loss_parts = pl.pallas_call(
        _rvq_kernel,
        grid=grid,
        in_specs=[
            pl.BlockSpec((1, D, TT), lambda j: (j // TPB, 0, j % TPB)),
            whole((NCB, CD, D)),
            whole((NCB, CD, 1)),
            whole((NCB, D, CD)),
            whole((NCB, D, 1)),
            whole((NCB, K, CD)),
            whole((NCB, CD, K)),
            whole((NCB, K, 1)),
        ],
        out_specs=[
            pl.BlockSpec((1, NCB, TT), lambda j: (j // TPB, 0, j % TPB)),
            pl.BlockSpec((1, D, TT), lambda j: (j // TPB, 0, j % TPB)),
            pl.BlockSpec((1, NCB, TT), lambda j: (j, 0, 0)),
        ],
        out_shape=[
            jax.ShapeDtypeStruct((B, NCB, T), jnp.int32),
            jax.ShapeDtypeStruct((B, D, T), f32),
            jax.ShapeDtypeStruct((B * TPB, NCB, TT), f32),
        ],
        compiler_params=pltpu.CompilerParams(
            dimension_semantics=("parallel",),
        ),
    )(z, w_in, b_in_r, w_out, b_out_r, cbn, cbt, c2)

    loss = jnp.sum(loss_parts) / f32(B * CD * T)
    return codes, latent, loss, loss


# speedup vs baseline: 3.1845x; 1.4822x over previous
import jax
import jax.numpy as jnp
from jax.experimental import pallas as pl
from jax.experimental.pallas import tpu as pltpu

B, D, T = 16, 1024, 1024
NCB, K, CD = 8, 1024, 64
TT = 512
TPB = T // TT


def _dot(a, b):
    return jax.lax.dot_general(a, b, (((1,), (0,)), ((), ())),
                               preferred_element_type=jnp.float32)


def _rvq_kernel(z_ref, w_in_ref, b_in_ref, w_out_ref, b_out_ref,
                cbn_ref, cbt_ref, c2_ref,
                codes_ref, latent_ref, loss_ref):
    z0 = z_ref[0]
    res = z0
    kiota = jax.lax.broadcasted_iota(jnp.int32, (K, TT), 0)
    for i in range(NCB):
        z_e = _dot(w_in_ref[i], res) + b_in_ref[i]
        acc = z_e * z_e
        h = CD
        while h > 1:
            h //= 2
            acc = acc[:h] + acc[h:]
        nrm = jnp.sqrt(acc)
        nrm = jnp.maximum(nrm, 1e-12)
        z_en = z_e / nrm
        s = _dot(cbn_ref[i], z_en)
        e = jnp.sum(z_en * z_en, axis=0, keepdims=True)
        dist = (e - 2.0 * s) + c2_ref[i]
        m = jnp.min(dist, axis=0, keepdims=True)
        hit = dist <= m
        idx = jnp.min(jnp.where(hit, kiota, K), axis=0, keepdims=True)
        oh = (kiota == idx).astype(jnp.float32)
        z_q = _dot(cbt_ref[i], oh)
        dq = z_e - z_q
        loss_ref[0, i:i + 1, :] = jnp.sum(dq * dq, axis=0, keepdims=True)
        z_q_out = _dot(w_out_ref[i], z_q) + b_out_ref[i]
        res = res - z_q_out
        codes_ref[0, i:i + 1, :] = idx
    latent_ref[0] = z0 - res


def kernel(z, v_in, g_in, b_in, v_out, g_out, b_out, codebooks):
    f32 = jnp.float32
    w_in_l, w_out_l, cbn_l, c2_l = [], [], [], []
    for i in range(NCB):
        wv = v_in[i]
        wnorm = jnp.sqrt(jnp.sum(wv * wv, axis=1, keepdims=True))
        w_in_l.append(g_in[i][:, None] * wv / wnorm)
        wo = v_out[i]
        wnorm_o = jnp.sqrt(jnp.sum(wo * wo, axis=1, keepdims=True))
        w_out_l.append(g_out[i][:, None] * wo / wnorm_o)
        cb = codebooks[i]
        cn = jnp.sqrt(jnp.sum(cb * cb, axis=1, keepdims=True))
        cbn_i = cb / jnp.maximum(cn, 1e-12)
        cbn_l.append(cbn_i)
        c2_l.append(jnp.sum(cbn_i * cbn_i, axis=1, keepdims=True))
    w_in = jnp.stack(w_in_l)
    w_out = jnp.stack(w_out_l)
    cbn = jnp.stack(cbn_l)
    c2 = jnp.stack(c2_l)
    cbt = jnp.swapaxes(codebooks, 1, 2)
    b_in_r = b_in[:, :, None]
    b_out_r = b_out[:, :, None]

    whole = lambda shape: pl.BlockSpec(shape, lambda j: (0,) * len(shape))
    grid = (B * TPB,)
    codes, latent, loss_parts = pl.pallas_call(
        _rvq_kernel,
        grid=grid,
        in_specs=[
            pl.BlockSpec((1, D, TT), lambda j: (j // TPB, 0, j % TPB)),
            whole((NCB, CD, D)),
            whole((NCB, CD, 1)),
            whole((NCB, D, CD)),
            whole((NCB, D, 1)),
            whole((NCB, K, CD)),
            whole((NCB, CD, K)),
            whole((NCB, K, 1)),
        ],
        out_specs=[
            pl.BlockSpec((1, NCB, TT), lambda j: (j // TPB, 0, j % TPB)),
            pl.BlockSpec((1, D, TT), lambda j: (j // TPB, 0, j % TPB)),
            pl.BlockSpec((1, NCB, TT), lambda j: (j, 0, 0)),
        ],
        out_shape=[
            jax.ShapeDtypeStruct((B, NCB, T), jnp.int32),
            jax.ShapeDtypeStruct((B, D, T), f32),
            jax.ShapeDtypeStruct((B * TPB, NCB, TT), f32),
        ],
        compiler_params=pltpu.CompilerParams(
            dimension_semantics=("parallel",),
        ),
    )(z, w_in, b_in_r, w_out, b_out_r, cbn, cbt, c2)

    loss = jnp.sum(loss_parts) / f32(B * CD * T)
    return codes, latent, loss, loss


# speedup vs baseline: 3.7205x; 1.1683x over previous
import jax
import jax.numpy as jnp
from jax.experimental import pallas as pl
from jax.experimental.pallas import tpu as pltpu

B, D, T = 16, 1024, 1024
NCB, K, CD = 8, 1024, 64
TT = 1024
TPB = T // TT


def _dot(a, b):
    return jax.lax.dot_general(a, b, (((1,), (0,)), ((), ())),
                               preferred_element_type=jnp.float32)


def _rvq_kernel(z_ref, w_in_ref, b_in_ref, w_out_ref, b_out_ref,
                cbn_ref, cbt_ref, c2_ref,
                codes_ref, latent_ref, loss_ref):
    z0 = z_ref[0]
    res = z0
    kiota = jax.lax.broadcasted_iota(jnp.int32, (K, TT), 0)
    for i in range(NCB):
        z_e = _dot(w_in_ref[i], res) + b_in_ref[i]
        acc = z_e * z_e
        h = CD
        while h > 1:
            h //= 2
            acc = acc[:h] + acc[h:]
        nrm = jnp.sqrt(acc)
        nrm = jnp.maximum(nrm, 1e-12)
        z_en = z_e / nrm
        s = _dot(cbn_ref[i], z_en)
        e = jnp.sum(z_en * z_en, axis=0, keepdims=True)
        dist = (e - 2.0 * s) + c2_ref[i]
        m = jnp.min(dist, axis=0, keepdims=True)
        hit = dist <= m
        idx = jnp.min(jnp.where(hit, kiota, K), axis=0, keepdims=True)
        oh = (kiota == idx).astype(jnp.float32)
        z_q = _dot(cbt_ref[i], oh)
        dq = z_e - z_q
        loss_ref[0, i:i + 1, :] = jnp.sum(dq * dq, axis=0, keepdims=True)
        z_q_out = _dot(w_out_ref[i], z_q) + b_out_ref[i]
        res = res - z_q_out
        codes_ref[0, i:i + 1, :] = idx
    latent_ref[0] = z0 - res


def kernel(z, v_in, g_in, b_in, v_out, g_out, b_out, codebooks):
    f32 = jnp.float32
    w_in_l, w_out_l, cbn_l, c2_l = [], [], [], []
    for i in range(NCB):
        wv = v_in[i]
        wnorm = jnp.sqrt(jnp.sum(wv * wv, axis=1, keepdims=True))
        w_in_l.append(g_in[i][:, None] * wv / wnorm)
        wo = v_out[i]
        wnorm_o = jnp.sqrt(jnp.sum(wo * wo, axis=1, keepdims=True))
        w_out_l.append(g_out[i][:, None] * wo / wnorm_o)
        cb = codebooks[i]
        cn = jnp.sqrt(jnp.sum(cb * cb, axis=1, keepdims=True))
        cbn_i = cb / jnp.maximum(cn, 1e-12)
        cbn_l.append(cbn_i)
        c2_l.append(jnp.sum(cbn_i * cbn_i, axis=1, keepdims=True))
    w_in = jnp.stack(w_in_l)
    w_out = jnp.stack(w_out_l)
    cbn = jnp.stack(cbn_l)
    c2 = jnp.stack(c2_l)
    cbt = jnp.swapaxes(codebooks, 1, 2)
    b_in_r = b_in[:, :, None]
    b_out_r = b_out[:, :, None]

    whole = lambda shape: pl.BlockSpec(shape, lambda j: (0,) * len(shape))
    grid = (B * TPB,)
    codes, latent, loss_parts = pl.pallas_call(
        _rvq_kernel,
        grid=grid,
        in_specs=[
            pl.BlockSpec((1, D, TT), lambda j: (j // TPB, 0, j % TPB)),
            whole((NCB, CD, D)),
            whole((NCB, CD, 1)),
            whole((NCB, D, CD)),
            whole((NCB, D, 1)),
            whole((NCB, K, CD)),
            whole((NCB, CD, K)),
            whole((NCB, K, 1)),
        ],
        out_specs=[
            pl.BlockSpec((1, NCB, TT), lambda j: (j // TPB, 0, j % TPB)),
            pl.BlockSpec((1, D, TT), lambda j: (j // TPB, 0, j % TPB)),
            pl.BlockSpec((1, NCB, TT), lambda j: (j, 0, 0)),
        ],
        out_shape=[
            jax.ShapeDtypeStruct((B, NCB, T), jnp.int32),
            jax.ShapeDtypeStruct((B, D, T), f32),
            jax.ShapeDtypeStruct((B * TPB, NCB, TT), f32),
        ],
        compiler_params=pltpu.CompilerParams(
            dimension_semantics=("parallel",),
        ),
    )(z, w_in, b_in_r, w_out, b_out_r, cbn, cbt, c2)

    loss = jnp.sum(loss_parts) / f32(B * CD * T)
    return codes, latent, loss, loss
